# Initial kernel scaffold; baseline (speedup 1.0000x reference)
import jax
import jax.numpy as jnp
from jax.experimental import pallas as pl


def kernel(W, beta, H, pred, Y, particle_id, track_params, reconstructable):
    raise NotImplementedError("write your pallas kernel here")


# speedup vs baseline: 1.0788x; 1.0788x over previous
import functools

import jax
import jax.numpy as jnp
from jax import lax
from jax.experimental import pallas as pl
from jax.experimental.pallas import tpu as pltpu
from jax.experimental.pallas import tpu_sc as plsc

N = 2_000_000
D = 5
NUM_P = 5000
P = 5120
NW = 32
CH = 1600
CH5 = CH * D
NCH = N // CH
KMAX = -(-NCH // NW)
GROUPS = CH // 16
STRIPS = P // 16


def _sc_body(pred_hbm, tp_hbm, pid_hbm, rec_hbm, mse_out, cnt_out,
             acc, pbuf, tbuf, ibuf, rbuf, red, sem):
    wid = lax.axis_index("s") * 2 + lax.axis_index("c")

    iota = lax.iota(jnp.int32, 16)
    iota5 = iota * 5
    lane_off = iota * P
    zero_v = jnp.zeros((16,), jnp.float32)
    one_v = jnp.ones((16,), jnp.float32)

    def zero_acc():
        def zb(s, carry):
            for u in range(8):
                acc[pl.ds((s * 8 + u) * 16, 16)] = zero_v
            return carry
        lax.fori_loop(0, (16 * P) // 128, zb, 0)

    def groups_mse(carry_unused):
        def gb(g, carry):
            b16 = g * 16
            pidv = ibuf[pl.ds(b16, 16)]
            recv = rbuf[pl.ds(b16, 16)]
            pid_eff = jnp.where(recv > 0, pidv, 0)
            fb = g * 80
            mse = zero_v
            for d in range(D):
                idx = iota5 + (fb + d)
                pv = plsc.load_gather(pbuf, [idx])
                tv = plsc.load_gather(tbuf, [idx])
                df = pv - tv
                mse = mse + df * df
            plsc.addupdate_scatter(acc, [lane_off + pid_eff], mse)
            return carry
        lax.fori_loop(0, GROUPS, gb, 0)

    def groups_cnt(carry_unused):
        def gb(g, carry):
            b16 = g * 16
            pidv = ibuf[pl.ds(b16, 16)]
            recv = rbuf[pl.ds(b16, 16)]
            pid_eff = jnp.where(recv > 0, pidv, 0)
            plsc.addupdate_scatter(acc, [lane_off + pid_eff], one_v)
            return carry
        lax.fori_loop(0, GROUPS, gb, 0)

    def chunk_loop(with_data, groups_fn):
        def kb(k, carry):
            c = wid + k * NW
            @pl.when(c < NCH)
            def _():
                cps = []
                if with_data:
                    cps.append(pltpu.async_copy(
                        pred_hbm.at[pl.ds(c * CH5, CH5)], pbuf, sem))
                    cps.append(pltpu.async_copy(
                        tp_hbm.at[pl.ds(c * CH5, CH5)], tbuf, sem))
                cps.append(pltpu.async_copy(
                    pid_hbm.at[pl.ds(c * CH, CH)], ibuf, sem))
                cps.append(pltpu.async_copy(
                    rec_hbm.at[pl.ds(c * CH, CH)], rbuf, sem))
                for cp in cps:
                    cp.wait()
                groups_fn(0)
            return carry
        lax.fori_loop(0, KMAX, kb, 0)

    def reduce_rows(out_ref):
        def rb(s, carry):
            col = s * 16
            v = acc[pl.ds(col, 16)]
            for r in range(1, 16):
                v = v + acc[pl.ds(r * P + col, 16)]
            red[pl.ds(col, 16)] = v
            return carry
        lax.fori_loop(0, STRIPS, rb, 0)
        pltpu.sync_copy(red, out_ref.at[wid])

    zero_acc()
    chunk_loop(True, groups_mse)
    reduce_rows(mse_out)
    zero_acc()
    chunk_loop(False, groups_cnt)
    reduce_rows(cnt_out)


_sc_segment = functools.partial(
    pl.kernel,
    out_type=(jax.ShapeDtypeStruct((NW, P), jnp.float32),
              jax.ShapeDtypeStruct((NW, P), jnp.float32)),
    mesh=plsc.VectorSubcoreMesh(core_axis_name="c", subcore_axis_name="s"),
    scratch_types=[
        pltpu.VMEM((16 * P,), jnp.float32),
        pltpu.VMEM((CH5,), jnp.float32),
        pltpu.VMEM((CH5,), jnp.float32),
        pltpu.VMEM((CH,), jnp.int32),
        pltpu.VMEM((CH,), jnp.int32),
        pltpu.VMEM((P,), jnp.float32),
        pltpu.SemaphoreType.DMA,
    ],
    compiler_params=pltpu.CompilerParams(needs_layout_passes=False),
)(_sc_body)


def _final_body(mse_ref, cnt_ref, out_ref):
    sum_mse = jnp.sum(mse_ref[...], axis=0, keepdims=True)
    counts = jnp.sum(cnt_ref[...], axis=0, keepdims=True)
    pids = lax.broadcasted_iota(jnp.int32, (1, P), 1).astype(jnp.float32)
    present = (counts > 0.0) & (pids != 0.0)
    xi_sum = pids * counts
    weighted = pids * sum_mse
    terms = jnp.where(present,
                      weighted / jnp.where(xi_sum > 0.0, xi_sum, 1.0),
                      0.0)
    k_cnt = jnp.sum(present.astype(jnp.float32))
    out_ref[0, 0] = 100.0 * jnp.sum(terms) / k_cnt


def kernel(W, beta, H, pred, Y, particle_id, track_params, reconstructable):
    pred_f = pred.reshape(-1)
    tp_f = track_params.reshape(-1)
    mse_part, cnt_part = _sc_segment(pred_f, tp_f, particle_id,
                                     reconstructable)
    out = pl.pallas_call(
        _final_body,
        out_shape=jax.ShapeDtypeStruct((1, 1), jnp.float32),
        out_specs=pl.BlockSpec(memory_space=pltpu.SMEM),
    )(mse_part, cnt_part)
    return out[0, 0]
